# Initial kernel scaffold; baseline (speedup 1.0000x reference)
import jax
import jax.numpy as jnp
from jax.experimental import pallas as pl


def kernel(anchors, deltas, logits):
    raise NotImplementedError("write your pallas kernel here")


# speedup vs baseline: 9.3855x; 9.3855x over previous
import functools
import math

import jax
import jax.numpy as jnp
from jax.experimental import pallas as pl
from jax.experimental.pallas import tpu as pltpu

_N_BOXES = 20000
_NUM_CLASSES = 80
_NUM_PREDS = 100
_IOU_THR = 0.5
_SCORE_THR = 0.3
_MAX_EDGE = 1024
_SCALE_CLAMP = math.log(1000.0 / 16)

_ROWS = 157
_LANES = 128
_NPAD = _ROWS * _LANES


def _body(l3_ref, ax1_ref, ay1_ref, ax2_ref, ay2_ref,
          dx_ref, dy_ref, dw_ref, dh_ref,
          pred_ref,
          x1c_ref, y1c_ref, x2c_ref, y2c_ref, catf_ref, area_ref):
    f32 = jnp.float32

    l = l3_ref[...]
    m = jnp.max(l, axis=2)
    ci = jax.lax.broadcasted_iota(jnp.int32, l.shape, 2)
    cat = jnp.min(jnp.where(l == m[..., None], ci, _NUM_CLASSES), axis=2)
    catf = cat.astype(f32)
    score = jax.nn.sigmoid(m)
    s0 = jnp.where(score >= _SCORE_THR, score, -1.0)

    ax1 = ax1_ref[...]
    ay1 = ay1_ref[...]
    ax2 = ax2_ref[...]
    ay2 = ay2_ref[...]
    widths = ax2 - ax1
    heights = ay2 - ay1
    ctr_x = (ax1 + ax2) * 0.5
    ctr_y = (ay1 + ay2) * 0.5
    dw = jnp.minimum(dw_ref[...], _SCALE_CLAMP)
    dh = jnp.minimum(dh_ref[...], _SCALE_CLAMP)
    pred_ctr_x = dx_ref[...] * widths + ctr_x
    pred_ctr_y = dy_ref[...] * heights + ctr_y
    pred_w = jnp.exp(dw) * widths
    pred_h = jnp.exp(dh) * heights
    hi = f32(_MAX_EDGE - 1.0)
    x1 = jnp.clip(pred_ctr_x - 0.5 * pred_w, 0.0, hi)
    y1 = jnp.clip(pred_ctr_y - 0.5 * pred_h, 0.0, hi)
    x2 = jnp.clip(pred_ctr_x + 0.5 * pred_w, 0.0, hi)
    y2 = jnp.clip(pred_ctr_y + 0.5 * pred_h, 0.0, hi)
    off = catf * f32(_MAX_EDGE)
    x1c = x1 + off
    y1c = y1 + off
    x2c = x2 + off
    y2c = y2 + off
    area = (x2c - x1c) * (y2c - y1c)

    x1c_ref[...] = x1c
    y1c_ref[...] = y1c
    x2c_ref[...] = x2c
    y2c_ref[...] = y2c
    catf_ref[...] = catf
    area_ref[...] = area

    flat_idx = (jax.lax.broadcasted_iota(jnp.int32, (_ROWS, _LANES), 0) * _LANES
                + jax.lax.broadcasted_iota(jnp.int32, (_ROWS, _LANES), 1))
    lane = jax.lax.broadcasted_iota(jnp.int32, (1, _LANES), 1)

    def nms_step(i, s):
        mx = jnp.max(s)
        j = jnp.min(jnp.where(s == mx, flat_idx, jnp.int32(2**30)))
        r = j // _LANES
        c = j - r * _LANES
        onehot = (lane == c).astype(f32)

        def ext(ref):
            return jnp.sum(ref[pl.ds(r, 1), :] * onehot)

        wx1 = ext(x1c_ref)
        wy1 = ext(y1c_ref)
        wx2 = ext(x2c_ref)
        wy2 = ext(y2c_ref)
        wcat = ext(catf_ref)
        warea = ext(area_ref)

        xx1 = jnp.maximum(wx1, x1c_ref[...])
        yy1 = jnp.maximum(wy1, y1c_ref[...])
        xx2 = jnp.minimum(wx2, x2c_ref[...])
        yy2 = jnp.minimum(wy2, y2c_ref[...])
        inter = jnp.maximum(xx2 - xx1, 0.0) * jnp.maximum(yy2 - yy1, 0.0)
        iou = inter / (warea + area_ref[...] - inter + 1e-9)
        suppress = (iou > _IOU_THR) | (flat_idx == j)
        s_new = jnp.where(suppress, -1.0, s)

        woff = wcat * f32(_MAX_EDGE)
        row = jnp.where(lane == 0, wcat,
              jnp.where(lane == 1, mx,
              jnp.where(lane == 2, wx1 - woff,
              jnp.where(lane == 3, wy1 - woff,
              jnp.where(lane == 4, wx2 - woff,
              jnp.where(lane == 5, wy2 - woff, -1.0))))))
        valid = mx > 0.0
        row = jnp.where(valid, row, -1.0)
        pred_ref[pl.ds(i, 1), :] = row
        return s_new

    jax.lax.fori_loop(0, _NUM_PREDS, nms_step, s0)


@jax.jit
def kernel(anchors, deltas, logits):
    f32 = jnp.float32
    pad = _NPAD - _N_BOXES

    l3 = jnp.pad(logits, ((0, pad), (0, 0)), constant_values=-100.0)
    l3 = l3.reshape(_ROWS, _LANES, _NUM_CLASSES)

    def col(a, k):
        return jnp.pad(a[:, k], (0, pad)).reshape(_ROWS, _LANES)

    ax1, ay1, ax2, ay2 = (col(anchors, k) for k in range(4))
    dx, dy, dw, dh = (col(deltas, k) for k in range(4))

    pred = pl.pallas_call(
        _body,
        out_shape=jax.ShapeDtypeStruct((_NUM_PREDS, _LANES), f32),
        in_specs=[pl.BlockSpec(memory_space=pltpu.VMEM)] * 9,
        out_specs=pl.BlockSpec(memory_space=pltpu.VMEM),
        scratch_shapes=[pltpu.VMEM((_ROWS, _LANES), f32)] * 6,
    )(l3, ax1, ay1, ax2, ay2, dx, dy, dw, dh)

    return pred[:, :6]


# speedup vs baseline: 9.7077x; 1.0343x over previous
import functools
import math

import jax
import jax.numpy as jnp
from jax.experimental import pallas as pl
from jax.experimental.pallas import tpu as pltpu

_N_BOXES = 20000
_NUM_CLASSES = 80
_NUM_PREDS = 100
_IOU_THR = 0.5
_SCORE_THR = 0.3
_MAX_EDGE = 1024
_SCALE_CLAMP = math.log(1000.0 / 16)

_ROWS = 157
_LANES = 128
_NPAD = _ROWS * _LANES


def _body(l3_ref, ax1_ref, ay1_ref, ax2_ref, ay2_ref,
          dx_ref, dy_ref, dw_ref, dh_ref,
          pred_ref,
          x1c_ref, y1c_ref, x2c_ref, y2c_ref, catf_ref, area_ref):
    f32 = jnp.float32

    p = jax.nn.sigmoid(l3_ref[...])
    m = jnp.max(p, axis=2)
    cat = jnp.argmax(p, axis=2)
    catf = cat.astype(f32)
    s0 = jnp.where(m >= _SCORE_THR, m, -1.0)

    ax1 = ax1_ref[...]
    ay1 = ay1_ref[...]
    ax2 = ax2_ref[...]
    ay2 = ay2_ref[...]
    widths = ax2 - ax1
    heights = ay2 - ay1
    ctr_x = (ax1 + ax2) * 0.5
    ctr_y = (ay1 + ay2) * 0.5
    dw = jnp.minimum(dw_ref[...], _SCALE_CLAMP)
    dh = jnp.minimum(dh_ref[...], _SCALE_CLAMP)
    pred_ctr_x = dx_ref[...] * widths + ctr_x
    pred_ctr_y = dy_ref[...] * heights + ctr_y
    pred_w = jnp.exp(dw) * widths
    pred_h = jnp.exp(dh) * heights
    hi = f32(_MAX_EDGE - 1.0)
    x1 = jnp.clip(pred_ctr_x - 0.5 * pred_w, 0.0, hi)
    y1 = jnp.clip(pred_ctr_y - 0.5 * pred_h, 0.0, hi)
    x2 = jnp.clip(pred_ctr_x + 0.5 * pred_w, 0.0, hi)
    y2 = jnp.clip(pred_ctr_y + 0.5 * pred_h, 0.0, hi)
    off = catf * f32(_MAX_EDGE)
    x1c = x1 + off
    y1c = y1 + off
    x2c = x2 + off
    y2c = y2 + off
    area = (x2c - x1c) * (y2c - y1c)

    x1c_ref[...] = x1c
    y1c_ref[...] = y1c
    x2c_ref[...] = x2c
    y2c_ref[...] = y2c
    catf_ref[...] = catf
    area_ref[...] = area

    flat_idx = (jax.lax.broadcasted_iota(jnp.int32, (_ROWS, _LANES), 0) * _LANES
                + jax.lax.broadcasted_iota(jnp.int32, (_ROWS, _LANES), 1))
    lane = jax.lax.broadcasted_iota(jnp.int32, (1, _LANES), 1)

    def nms_step(i, carry):
        s, mx = carry
        j = jnp.min(jnp.where(s == mx, flat_idx, jnp.int32(2**30)))
        r = j // _LANES
        c = j - r * _LANES
        onehot = (lane == c).astype(f32)

        def ext(ref):
            return jnp.sum(ref[pl.ds(r, 1), :] * onehot)

        wx1 = ext(x1c_ref)
        wy1 = ext(y1c_ref)
        wx2 = ext(x2c_ref)
        wy2 = ext(y2c_ref)
        wcat = ext(catf_ref)
        warea = ext(area_ref)

        xx1 = jnp.maximum(wx1, x1c_ref[...])
        yy1 = jnp.maximum(wy1, y1c_ref[...])
        xx2 = jnp.minimum(wx2, x2c_ref[...])
        yy2 = jnp.minimum(wy2, y2c_ref[...])
        inter = jnp.maximum(xx2 - xx1, 0.0) * jnp.maximum(yy2 - yy1, 0.0)
        iou = inter / (warea + area_ref[...] - inter + 1e-9)
        suppress = (iou > _IOU_THR) | (flat_idx == j)
        s_new = jnp.where(suppress, -1.0, s)
        mx_new = jnp.max(s_new)

        woff = wcat * f32(_MAX_EDGE)
        row = jnp.where(lane == 0, wcat,
              jnp.where(lane == 1, mx,
              jnp.where(lane == 2, wx1 - woff,
              jnp.where(lane == 3, wy1 - woff,
              jnp.where(lane == 4, wx2 - woff,
              jnp.where(lane == 5, wy2 - woff, -1.0))))))
        valid = mx > 0.0
        row = jnp.where(valid, row, -1.0)
        pred_ref[pl.ds(i, 1), :] = row
        return (s_new, mx_new)

    jax.lax.fori_loop(0, _NUM_PREDS, nms_step, (s0, jnp.max(s0)))


@jax.jit
def kernel(anchors, deltas, logits):
    f32 = jnp.float32
    pad = _NPAD - _N_BOXES

    l3 = jnp.pad(logits, ((0, pad), (0, 0)), constant_values=-100.0)
    l3 = l3.reshape(_ROWS, _LANES, _NUM_CLASSES)

    def col(a, k):
        return jnp.pad(a[:, k], (0, pad)).reshape(_ROWS, _LANES)

    ax1, ay1, ax2, ay2 = (col(anchors, k) for k in range(4))
    dx, dy, dw, dh = (col(deltas, k) for k in range(4))

    pred = pl.pallas_call(
        _body,
        out_shape=jax.ShapeDtypeStruct((_NUM_PREDS, _LANES), f32),
        in_specs=[pl.BlockSpec(memory_space=pltpu.VMEM)] * 9,
        out_specs=pl.BlockSpec(memory_space=pltpu.VMEM),
        scratch_shapes=[pltpu.VMEM((_ROWS, _LANES), f32)] * 6,
    )(l3, ax1, ay1, ax2, ay2, dx, dy, dw, dh)

    return pred[:, :6]


# speedup vs baseline: 14.4846x; 1.4921x over previous
import functools
import math

import jax
import jax.numpy as jnp
from jax.experimental import pallas as pl
from jax.experimental.pallas import tpu as pltpu

_N_BOXES = 20000
_NUM_CLASSES = 80
_NUM_PREDS = 100
_IOU_THR = 0.5
_SCORE_THR = 0.3
_MAX_EDGE = 1024
_SCALE_CLAMP = math.log(1000.0 / 16)

_ROWS = 157
_LANES = 128
_NPAD = _ROWS * _LANES


def _body(l3_ref, ax1_ref, ay1_ref, ax2_ref, ay2_ref,
          dx_ref, dy_ref, dw_ref, dh_ref,
          pred_ref,
          x1c_ref, y1c_ref, x2c_ref, y2c_ref, catf_ref, area_ref):
    f32 = jnp.float32

    l = jnp.concatenate(
        [l3_ref[...],
         jnp.full((_NPAD - _N_BOXES, _NUM_CLASSES), -100.0, jnp.float32)],
        axis=0).reshape(_ROWS, _LANES, _NUM_CLASSES)
    p = jax.nn.sigmoid(l)
    m = jnp.max(p, axis=2)
    cat = jnp.argmax(p, axis=2)
    catf = cat.astype(f32)
    s0 = jnp.where(m >= _SCORE_THR, m, -1.0)

    ax1 = ax1_ref[...]
    ay1 = ay1_ref[...]
    ax2 = ax2_ref[...]
    ay2 = ay2_ref[...]
    widths = ax2 - ax1
    heights = ay2 - ay1
    ctr_x = (ax1 + ax2) * 0.5
    ctr_y = (ay1 + ay2) * 0.5
    dw = jnp.minimum(dw_ref[...], _SCALE_CLAMP)
    dh = jnp.minimum(dh_ref[...], _SCALE_CLAMP)
    pred_ctr_x = dx_ref[...] * widths + ctr_x
    pred_ctr_y = dy_ref[...] * heights + ctr_y
    pred_w = jnp.exp(dw) * widths
    pred_h = jnp.exp(dh) * heights
    hi = f32(_MAX_EDGE - 1.0)
    x1 = jnp.clip(pred_ctr_x - 0.5 * pred_w, 0.0, hi)
    y1 = jnp.clip(pred_ctr_y - 0.5 * pred_h, 0.0, hi)
    x2 = jnp.clip(pred_ctr_x + 0.5 * pred_w, 0.0, hi)
    y2 = jnp.clip(pred_ctr_y + 0.5 * pred_h, 0.0, hi)
    off = catf * f32(_MAX_EDGE)
    x1c = x1 + off
    y1c = y1 + off
    x2c = x2 + off
    y2c = y2 + off
    area = (x2c - x1c) * (y2c - y1c)

    x1c_ref[...] = x1c
    y1c_ref[...] = y1c
    x2c_ref[...] = x2c
    y2c_ref[...] = y2c
    catf_ref[...] = catf
    area_ref[...] = area

    flat_idx = (jax.lax.broadcasted_iota(jnp.int32, (_ROWS, _LANES), 0) * _LANES
                + jax.lax.broadcasted_iota(jnp.int32, (_ROWS, _LANES), 1))
    lane = jax.lax.broadcasted_iota(jnp.int32, (1, _LANES), 1)

    def nms_step(i, carry):
        s, mx = carry
        j = jnp.min(jnp.where(s == mx, flat_idx, jnp.int32(2**30)))
        r = j // _LANES
        c = j - r * _LANES
        onehot = (lane == c).astype(f32)

        def ext(ref):
            return jnp.sum(ref[pl.ds(r, 1), :] * onehot)

        wx1 = ext(x1c_ref)
        wy1 = ext(y1c_ref)
        wx2 = ext(x2c_ref)
        wy2 = ext(y2c_ref)
        wcat = ext(catf_ref)
        warea = ext(area_ref)

        xx1 = jnp.maximum(wx1, x1c_ref[...])
        yy1 = jnp.maximum(wy1, y1c_ref[...])
        xx2 = jnp.minimum(wx2, x2c_ref[...])
        yy2 = jnp.minimum(wy2, y2c_ref[...])
        inter = jnp.maximum(xx2 - xx1, 0.0) * jnp.maximum(yy2 - yy1, 0.0)
        iou = inter / (warea + area_ref[...] - inter + 1e-9)
        suppress = (iou > _IOU_THR) | (flat_idx == j)
        s_new = jnp.where(suppress, -1.0, s)
        mx_new = jnp.max(s_new)

        woff = wcat * f32(_MAX_EDGE)
        row = jnp.where(lane == 0, wcat,
              jnp.where(lane == 1, mx,
              jnp.where(lane == 2, wx1 - woff,
              jnp.where(lane == 3, wy1 - woff,
              jnp.where(lane == 4, wx2 - woff,
              jnp.where(lane == 5, wy2 - woff, -1.0))))))
        valid = mx > 0.0
        row = jnp.where(valid, row, -1.0)
        pred_ref[pl.ds(i, 1), :] = row
        return (s_new, mx_new)

    jax.lax.fori_loop(0, _NUM_PREDS, nms_step, (s0, jnp.max(s0)))


@jax.jit
def kernel(anchors, deltas, logits):
    f32 = jnp.float32
    pad = _NPAD - _N_BOXES

    def col(a, k):
        return jnp.pad(a[:, k], (0, pad)).reshape(_ROWS, _LANES)

    ax1, ay1, ax2, ay2 = (col(anchors, k) for k in range(4))
    dx, dy, dw, dh = (col(deltas, k) for k in range(4))

    pred = pl.pallas_call(
        _body,
        out_shape=jax.ShapeDtypeStruct((_NUM_PREDS, _LANES), f32),
        in_specs=[pl.BlockSpec(memory_space=pltpu.VMEM)] * 9,
        out_specs=pl.BlockSpec(memory_space=pltpu.VMEM),
        scratch_shapes=[pltpu.VMEM((_ROWS, _LANES), f32)] * 6,
    )(logits, ax1, ay1, ax2, ay2, dx, dy, dw, dh)

    return pred[:, :6]
